# Initial kernel scaffold; baseline (speedup 1.0000x reference)
import jax
import jax.numpy as jnp
from jax.experimental import pallas as pl


def kernel(x, edge_index, edge_attr, Wn, bn, We, be):
    raise NotImplementedError("write your pallas kernel here")


# speedup vs baseline: 1.5703x; 1.5703x over previous
import functools

import jax
import jax.numpy as jnp
from jax import lax
from jax.experimental import pallas as pl
from jax.experimental.pallas import tpu as pltpu
from jax.experimental.pallas import tpu_sc as plsc

N = 10000
E = 160000
D = 256
DE = 16
L = 3

NC = 2
NS = 16
HALF = D // NC
CH = 128
E_PAD = 163840
NCHUNK = E_PAD // CH
CPS = NCHUNK // NS
N_PAD = 10240
ROWS_PER_SUB = N_PAD // NS
AGG_ROWS = N_PAD


def _ee_body(ea_ref, we_ref, be_ref, out_ref):
    ea = ea_ref[...]
    for l in range(L):
        for c in range(NC):
            o = jnp.dot(ea, we_ref[l, c], preferred_element_type=jnp.float32)
            out_ref[l, c] = o + be_ref[l, c][None, :]


def _ee_all(ea_p, We_r, be_r):
    BE = 2048
    return pl.pallas_call(
        _ee_body,
        grid=(E_PAD // BE,),
        in_specs=[
            pl.BlockSpec((BE, DE), lambda i: (i, 0)),
            pl.BlockSpec((L, NC, DE, HALF), lambda i: (0, 0, 0, 0)),
            pl.BlockSpec((L, NC, HALF), lambda i: (0, 0, 0)),
        ],
        out_specs=pl.BlockSpec((L, NC, BE, HALF), lambda i: (0, 0, i, 0)),
        out_shape=jax.ShapeDtypeStruct((L, NC, E_PAD, HALF), jnp.float32),
    )(ea_p, We_r, be_r)


def _mm_body(agg_ref, wn_ref, bn_ref, out_ref):
    a0 = agg_ref[0]
    a1 = agg_ref[1]
    for c in range(NC):
        o = (jnp.dot(a0, wn_ref[0, c], preferred_element_type=jnp.float32)
             + jnp.dot(a1, wn_ref[1, c], preferred_element_type=jnp.float32)
             + bn_ref[c][None, :])
        out_ref[c] = jnp.where(o > 0, o, 0.01 * o)


def _mm_layer(agg, Wn_l, bn_l):
    BN = 1024
    return pl.pallas_call(
        _mm_body,
        grid=(N_PAD // BN,),
        in_specs=[
            pl.BlockSpec((NC, BN, HALF), lambda i: (0, i, 0)),
            pl.BlockSpec((NC, NC, HALF, HALF), lambda i: (0, 0, 0, 0)),
            pl.BlockSpec((NC, HALF), lambda i: (0, 0)),
        ],
        out_specs=pl.BlockSpec((NC, BN, HALF), lambda i: (0, i, 0)),
        out_shape=jax.ShapeDtypeStruct((NC, N_PAD, HALF), jnp.float32),
    )(agg, Wn_l, bn_l)


def _sc_layer(l, h_split, ee_all, src2d, dst2d):
    mesh = plsc.VectorSubcoreMesh(core_axis_name="c", subcore_axis_name="s")

    @functools.partial(
        pl.kernel,
        mesh=mesh,
        out_type=jax.ShapeDtypeStruct((NC, N_PAD, HALF), jnp.float32),
        scratch_types=[
            pltpu.VMEM((CH,), jnp.int32),
            pltpu.VMEM((CH,), jnp.int32),
            pltpu.VMEM((CH, HALF), jnp.float32),
            pltpu.VMEM((CH, HALF), jnp.float32),
            pltpu.VMEM_SHARED((AGG_ROWS, HALF), jnp.float32),
            pltpu.SemaphoreType.DMA,
        ],
    )
    def k(h_hbm, ee_hbm, src_hbm, dst_hbm, out_hbm,
          src_v, dst_v, rows_v, ee_v, agg_sh, sem):
        c = lax.axis_index("c")
        s = lax.axis_index("s")
        base_r = s * ROWS_PER_SUB
        pltpu.sync_copy(h_hbm.at[c, pl.ds(base_r, ROWS_PER_SUB)],
                        agg_sh.at[pl.ds(base_r, ROWS_PER_SUB)])
        plsc.subcore_barrier()

        def chunk_body(i, carry):
            g = s * CPS + i
            pltpu.sync_copy(src_hbm.at[g], src_v)
            pltpu.sync_copy(dst_hbm.at[g], dst_v)
            pltpu.async_copy(h_hbm.at[c].at[src_v], rows_v, sem).wait()
            pltpu.sync_copy(ee_hbm.at[l, c, pl.ds(g * CH, CH)], ee_v)

            def row_body(r, rcarry):
                for j in range(HALF // 16):
                    sl = pl.ds(j * 16, 16)
                    v = rows_v[r, sl] + ee_v[r, sl]
                    rows_v[r, sl] = jnp.maximum(v, 0.0)
                return rcarry

            lax.fori_loop(0, CH, row_body, 0)
            pltpu.sync_copy(rows_v, agg_sh.at[dst_v], add=True)
            return carry

        lax.fori_loop(0, CPS, chunk_body, 0)
        plsc.subcore_barrier()
        pltpu.sync_copy(agg_sh.at[pl.ds(base_r, ROWS_PER_SUB)],
                        out_hbm.at[c, pl.ds(base_r, ROWS_PER_SUB)])

    return k(h_split, ee_all, src2d, dst2d)


def kernel(x, edge_index, edge_attr, Wn, bn, We, be):
    pad = E_PAD - E
    src_p = jnp.concatenate([edge_index[0].astype(jnp.int32),
                             jnp.zeros((pad,), jnp.int32)])
    dst_p = jnp.concatenate([edge_index[1].astype(jnp.int32),
                             N + (jnp.arange(pad, dtype=jnp.int32) % 16)])
    src2d = src_p.reshape(NCHUNK, CH)
    dst2d = dst_p.reshape(NCHUNK, CH)
    ea_p = jnp.concatenate([edge_attr,
                            jnp.zeros((pad, DE), edge_attr.dtype)])
    We_r = We.reshape(L, DE, NC, HALF).transpose(0, 2, 1, 3)
    be_r = be.reshape(L, NC, HALF)
    Wn_r = Wn.reshape(L, NC, HALF, NC, HALF).transpose(0, 1, 3, 2, 4)
    bn_r = bn.reshape(L, NC, HALF)
    x_p = jnp.concatenate([x, jnp.zeros((N_PAD - N, D), x.dtype)])
    h = x_p.reshape(N_PAD, NC, HALF).transpose(1, 0, 2)

    ee_all = _ee_all(ea_p, We_r, be_r)

    for l in range(L):
        agg = _sc_layer(l, h, ee_all, src2d, dst2d)
        h = _mm_layer(agg, Wn_r[l], bn_r[l])
    return h.transpose(1, 0, 2).reshape(N_PAD, D)[:N]
